# baseline (speedup 1.0000x reference)
import jax
import jax.numpy as jnp
from jax.experimental import pallas as pl
from jax.experimental.pallas import tpu as pltpu

BLOCK_M = 256
CHUNK_K = 256
DEPTH = 8


def _gate_kernel(x_hbm, w1_ref, b1_ref, w2_ref, b2_ref,
                 weight_ref, soft_ref, bufs, sems):
    M, K = x_hbm.shape
    nblk = M // BLOCK_M
    n = soft_ref.shape[-1]
    w1 = w1_ref[...].astype(jnp.bfloat16)
    w2 = w2_ref[...].astype(jnp.bfloat16)
    b1 = b1_ref[...]
    b2 = b2_ref[...]

    def start_copy(i, slot):
        pltpu.make_async_copy(
            x_hbm.at[pl.ds(i * BLOCK_M, BLOCK_M), :],
            bufs.at[slot],
            sems.at[slot],
        ).start()

    for d in range(DEPTH - 1):
        start_copy(d, d)

    def step(i, carry):
        slot = jax.lax.rem(i, DEPTH)
        nxt = i + DEPTH - 1

        @pl.when(nxt < nblk)
        def _():
            start_copy(nxt, jax.lax.rem(nxt, DEPTH))

        pltpu.make_async_copy(
            x_hbm.at[pl.ds(i * BLOCK_M, BLOCK_M), :],
            bufs.at[slot],
            sems.at[slot],
        ).wait()

        xb = bufs[slot].astype(jnp.bfloat16)
        acc = jnp.dot(xb[:, 0:CHUNK_K], w1[0:CHUNK_K, :],
                      preferred_element_type=jnp.float32)
        for k0 in range(CHUNK_K, K, CHUNK_K):
            acc = acc + jnp.dot(xb[:, k0:k0 + CHUNK_K], w1[k0:k0 + CHUNK_K, :],
                                preferred_element_type=jnp.float32)
        h = jnp.maximum(acc + b1, 0.0)
        logits = jnp.dot(h.astype(jnp.bfloat16), w2,
                         preferred_element_type=jnp.float32) + b2

        m = jnp.max(logits, axis=-1, keepdims=True)
        e = jnp.exp(logits - m)
        t = e[:, 0:8] + e[:, 8:16]
        t = t[:, 0:4] + t[:, 4:8]
        t = t[:, 0:2] + t[:, 2:4]
        s = t[:, 0:1] + t[:, 1:2]
        soft = e / s
        soft_ref[pl.ds(i * BLOCK_M, BLOCK_M), :] = soft

        iota = jax.lax.broadcasted_iota(jnp.int32, soft.shape, 1)
        sm = jnp.max(soft, axis=-1, keepdims=True)
        first = jnp.min(jnp.where(soft == sm, iota, n), axis=-1, keepdims=True)
        weight_ref[pl.ds(i * BLOCK_M, BLOCK_M), :] = (
            (iota == first).astype(jnp.float32))
        return carry

    jax.lax.fori_loop(0, nblk, step, 0)


@jax.jit
def kernel(x, W1, b1, W2, b2):
    M, K = x.shape
    H = W1.shape[1]
    N = W2.shape[1]
    weight, soft = pl.pallas_call(
        _gate_kernel,
        in_specs=[
            pl.BlockSpec(memory_space=pltpu.HBM),
            pl.BlockSpec(memory_space=pltpu.VMEM),
            pl.BlockSpec(memory_space=pltpu.VMEM),
            pl.BlockSpec(memory_space=pltpu.VMEM),
            pl.BlockSpec(memory_space=pltpu.VMEM),
        ],
        out_specs=[
            pl.BlockSpec(memory_space=pltpu.VMEM),
            pl.BlockSpec(memory_space=pltpu.VMEM),
        ],
        out_shape=[
            jax.ShapeDtypeStruct((M, N), jnp.float32),
            jax.ShapeDtypeStruct((M, N), jnp.float32),
        ],
        scratch_shapes=[
            pltpu.VMEM((DEPTH, BLOCK_M, K), jnp.float32),
            pltpu.SemaphoreType.DMA((DEPTH,)),
        ],
    )(x, W1, b1.reshape(1, H), W2, b2.reshape(1, N))
    return (weight, soft)


# speedup vs baseline: 1.1485x; 1.1485x over previous
import jax
import jax.numpy as jnp
from jax.experimental import pallas as pl
from jax.experimental.pallas import tpu as pltpu

BLOCK_M = 512
CHUNK_K = 256
DEPTH = 4


def _gate_kernel(x_hbm, w1_ref, b1_ref, w2_ref, b2_ref,
                 weight_ref, soft_ref, bufs, lring, sems):
    M, K = x_hbm.shape
    nblk = M // BLOCK_M
    n = soft_ref.shape[-1]
    w1 = w1_ref[...].astype(jnp.bfloat16)
    w2 = w2_ref[...].astype(jnp.bfloat16)
    b1 = b1_ref[...]
    b2 = b2_ref[...]

    def start_copy(i, slot):
        pltpu.make_async_copy(
            x_hbm.at[pl.ds(i * BLOCK_M, BLOCK_M), :],
            bufs.at[slot],
            sems.at[slot],
        ).start()

    for d in range(DEPTH - 1):
        start_copy(d, d)

    def step(i, carry):
        @pl.when(i < nblk)
        def _dots():
            slot = jax.lax.rem(i, DEPTH)
            nxt = i + DEPTH - 1

            @pl.when(nxt < nblk)
            def _():
                start_copy(nxt, jax.lax.rem(nxt, DEPTH))

            pltpu.make_async_copy(
                x_hbm.at[pl.ds(i * BLOCK_M, BLOCK_M), :],
                bufs.at[slot],
                sems.at[slot],
            ).wait()

            xb = bufs[slot].astype(jnp.bfloat16)
            acc = jnp.dot(xb[:, 0:CHUNK_K], w1[0:CHUNK_K, :],
                          preferred_element_type=jnp.float32)
            for k0 in range(CHUNK_K, K, CHUNK_K):
                acc = acc + jnp.dot(xb[:, k0:k0 + CHUNK_K],
                                    w1[k0:k0 + CHUNK_K, :],
                                    preferred_element_type=jnp.float32)
            h = jnp.maximum(acc + b1, 0.0)
            lring[jax.lax.rem(i, 2)] = jnp.dot(
                h.astype(jnp.bfloat16), w2,
                preferred_element_type=jnp.float32) + b2

        @pl.when(i > 0)
        def _tail():
            j = i - 1
            logits = lring[jax.lax.rem(j, 2)]
            m = jnp.max(logits, axis=-1, keepdims=True)
            e = jnp.exp(logits - m)
            t = e[:, 0:8] + e[:, 8:16]
            t = t[:, 0:4] + t[:, 4:8]
            t = t[:, 0:2] + t[:, 2:4]
            s = t[:, 0:1] + t[:, 1:2]
            soft = e / s
            soft_ref[pl.ds(j * BLOCK_M, BLOCK_M), :] = soft

            iota = jax.lax.broadcasted_iota(jnp.int32, soft.shape, 1)
            sm = jnp.max(soft, axis=-1, keepdims=True)
            first = jnp.min(jnp.where(soft == sm, iota, n),
                            axis=-1, keepdims=True)
            weight_ref[pl.ds(j * BLOCK_M, BLOCK_M), :] = (
                (iota == first).astype(jnp.float32))

        return carry

    jax.lax.fori_loop(0, nblk + 1, step, 0)


@jax.jit
def kernel(x, W1, b1, W2, b2):
    M, K = x.shape
    H = W1.shape[1]
    N = W2.shape[1]
    weight, soft = pl.pallas_call(
        _gate_kernel,
        in_specs=[
            pl.BlockSpec(memory_space=pltpu.HBM),
            pl.BlockSpec(memory_space=pltpu.VMEM),
            pl.BlockSpec(memory_space=pltpu.VMEM),
            pl.BlockSpec(memory_space=pltpu.VMEM),
            pl.BlockSpec(memory_space=pltpu.VMEM),
        ],
        out_specs=[
            pl.BlockSpec(memory_space=pltpu.VMEM),
            pl.BlockSpec(memory_space=pltpu.VMEM),
        ],
        out_shape=[
            jax.ShapeDtypeStruct((M, N), jnp.float32),
            jax.ShapeDtypeStruct((M, N), jnp.float32),
        ],
        scratch_shapes=[
            pltpu.VMEM((DEPTH, BLOCK_M, K), jnp.float32),
            pltpu.VMEM((2, BLOCK_M, N), jnp.float32),
            pltpu.SemaphoreType.DMA((DEPTH,)),
        ],
    )(x, W1, b1.reshape(1, H), W2, b2.reshape(1, N))
    return (weight, soft)
